# Initial kernel scaffold; baseline (speedup 1.0000x reference)
import jax
import jax.numpy as jnp
from jax.experimental import pallas as pl


def kernel(adj_indices, adj_values, user_emb, item_emb):
    raise NotImplementedError("write your pallas kernel here")


# speedup vs baseline: 3.4337x; 3.4337x over previous
import functools

import jax
import jax.numpy as jnp
from jax import lax
from jax.experimental import pallas as pl
from jax.experimental.pallas import tpu as pltpu
from jax.experimental.pallas import tpu_sc as plsc

NUM_USERS = 25000
NUM_ITEMS = 25000
DIM = 64
NUM_LAYERS = 3
E = 800000
N = NUM_USERS + NUM_ITEMS

NC = 2
NS = 16
L = 16

K = 128
CHUNKS = -(-E // (NS * K))
EPT = CHUNKS * K
EP = EPT * NS
NPT = N // NS
RW = 125
HD = DIM // 2


def _layer_body(mult, table, rowp, colp, valp, fin, zrows, newt, fout,
                acc, ridx, cidx, vals_v, rows, abuf, fbuf, gsem):
    c = lax.axis_index("c")
    s = lax.axis_index("s")
    cn = c * N
    r0 = s * NPT

    pltpu.sync_copy(zrows, acc.at[pl.ds(r0, NPT)])
    plsc.subcore_barrier()

    base = s * EPT

    def chunk(g, carry):
        off = base + g * K
        pltpu.sync_copy(rowp.at[pl.ds(off, K)], ridx)
        pltpu.sync_copy(colp.at[pl.ds(off, K)], cidx)
        pltpu.sync_copy(valp.at[pl.ds(off, K)], vals_v)

        def adj(i, _):
            cidx[pl.ds(i * L, L)] = cidx[pl.ds(i * L, L)] + cn
            return 0
        lax.fori_loop(0, K // L, adj, 0, unroll=True)

        pltpu.async_copy(table.at[cidx], rows, gsem).wait()

        def scale(g, _):
            v = vals_v[pl.ds(g * L, L)]
            for i in range(L):
                e = g * L + i
                b = v[i]
                rows[e, pl.ds(0, L)] = rows[e, pl.ds(0, L)] * b
                rows[e, pl.ds(L, L)] = rows[e, pl.ds(L, L)] * b
            return 0
        lax.fori_loop(0, K // L, scale, 0)

        pltpu.sync_copy(rows, acc.at[ridx], add=True)
        return carry

    lax.fori_loop(0, CHUNKS, chunk, 0)
    plsc.subcore_barrier()

    def wb(b, _):
        rb = r0 + b * RW
        pltpu.sync_copy(acc.at[pl.ds(rb, RW)], abuf)
        pltpu.sync_copy(fin.at[pl.ds(cn + rb, RW)], fbuf)

        def addrow(i, _):
            for j in (0, L):
                fbuf[i, pl.ds(j, L)] = (
                    fbuf[i, pl.ds(j, L)] + abuf[i, pl.ds(j, L)]) * mult
            return 0
        lax.fori_loop(0, RW, addrow, 0)

        pltpu.sync_copy(abuf, newt.at[pl.ds(cn + rb, RW)])
        pltpu.sync_copy(fbuf, fout.at[pl.ds(cn + rb, RW)])
        return 0
    lax.fori_loop(0, NPT // RW, wb, 0)


def _make_layer(mult):
    mesh = plsc.VectorSubcoreMesh(core_axis_name="c", subcore_axis_name="s")
    return pl.kernel(
        functools.partial(_layer_body, mult),
        out_type=(
            jax.ShapeDtypeStruct((NC * N, HD), jnp.float32),
            jax.ShapeDtypeStruct((NC * N, HD), jnp.float32),
        ),
        mesh=mesh,
        compiler_params=pltpu.CompilerParams(use_tc_tiling_on_sc=False),
        scratch_types=[
            pltpu.VMEM_SHARED((N, HD), jnp.float32),
            pltpu.VMEM((K,), jnp.int32),
            pltpu.VMEM((K,), jnp.int32),
            pltpu.VMEM((K,), jnp.float32),
            pltpu.VMEM((K, HD), jnp.float32),
            pltpu.VMEM((RW, HD), jnp.float32),
            pltpu.VMEM((RW, HD), jnp.float32),
            pltpu.SemaphoreType.DMA,
        ],
        name=f"lightgcn_layer_m{int(mult * 100)}",
    )


def kernel(adj_indices, adj_values, user_emb, item_emb):
    alpha = 1.0 / (NUM_LAYERS + 1)
    all_embs = jnp.concatenate([user_emb, item_emb], axis=0)
    t0 = all_embs.reshape(N, NC, HD).transpose(1, 0, 2).reshape(NC * N, HD)

    pad = EP - E
    rowp = jnp.concatenate([adj_indices[0], jnp.zeros((pad,), jnp.int32)])
    colp = jnp.concatenate([adj_indices[1], jnp.zeros((pad,), jnp.int32)])
    valp = jnp.concatenate([adj_values, jnp.zeros((pad,), jnp.float32)])
    zrows = jnp.zeros((NPT, HD), jnp.float32)

    step1 = _make_layer(1.0)
    step3 = _make_layer(alpha)

    t1, f1 = step1(t0, rowp, colp, valp, t0, zrows)
    t2, f2 = step1(t1, rowp, colp, valp, f1, zrows)
    _, f3 = step3(t2, rowp, colp, valp, f2, zrows)

    final = f3.reshape(NC, N, HD).transpose(1, 0, 2).reshape(N, DIM)
    return final[:NUM_USERS], final[NUM_USERS:]
